# Initial kernel scaffold; baseline (speedup 1.0000x reference)
import functools

import jax
import jax.numpy as jnp
from jax import lax
from jax.experimental import pallas as pl
from jax.experimental.pallas import tpu as pltpu
from jax.experimental.pallas import tpu_sc as plsc


def kernel(x, table, hash_coeffs):
    rows, n_chunks, chunk_size = table.shape
    bf = x.shape[0] * x.shape[1]
    r_total = bf * n_chunks

    NC, NS, L = 2, 16, 16
    NW = NC * NS
    T = 2048
    per_w = r_total // NW
    G = per_w // T
    IDS = T // n_chunks

    xf = x.reshape(-1).astype(jnp.int32)
    cm = (hash_coeffs % rows).astype(jnp.int32)
    c_hi = cm // 1000
    c_lo = cm % 1000
    coef = jnp.broadcast_to(
        jnp.concatenate([c_hi, c_lo]).reshape(2 * n_chunks, 1),
        (2 * n_chunks, L)).astype(jnp.int32)
    tab = table.reshape(rows * n_chunks, chunk_size)

    mesh = plsc.VectorSubcoreMesh(core_axis_name="c", subcore_axis_name="s")

    @functools.partial(
        pl.kernel, mesh=mesh,
        out_type=jax.ShapeDtypeStruct((r_total, chunk_size), jnp.float32),
        scratch_types=[
            pltpu.VMEM((IDS,), jnp.int32),
            pltpu.VMEM((2 * n_chunks, L), jnp.int32),
            pltpu.VMEM((T,), jnp.int32),
            pltpu.VMEM((T, chunk_size), jnp.float32),
            pltpu.SemaphoreType.DMA,
        ],
    )
    def sc_kern(xf_hbm, coef_hbm, tab_hbm, out_hbm,
                x_v, coef_v, idx_v, rows_v, sem):
        wid = lax.axis_index("s") * NC + lax.axis_index("c")
        pltpu.sync_copy(coef_hbm, coef_v)
        base_r = wid * per_w
        lane = lax.iota(jnp.int32, L)

        def step(g, carry):
            r0 = base_r + g * T
            bf0 = r0 // n_chunks
            pltpu.sync_copy(xf_hbm.at[pl.ds(bf0, IDS)], x_v)

            def hash_step(i, c2):
                xv = x_v[pl.ds(i * L, L)]
                pos0 = i * (L * n_chunks) + lane * n_chunks
                for c in range(n_chunks):
                    chi_v = coef_v[c]
                    clo_v = coef_v[n_chunks + c]
                    h = ((xv * chi_v) % rows * 1000 + xv * clo_v) % rows
                    plsc.store_scatter(idx_v, [pos0 + c], h * n_chunks + c)
                return c2

            lax.fori_loop(0, IDS // L, hash_step, 0)

            copies = []
            for j in range(T // 128):
                copies.append(pltpu.async_copy(
                    tab_hbm.at[idx_v.at[pl.ds(j * 128, 128)]],
                    rows_v.at[pl.ds(j * 128, 128)],
                    sem))
            for cp in copies:
                cp.wait()
            pltpu.sync_copy(rows_v, out_hbm.at[pl.ds(r0, T)])
            return carry

        lax.fori_loop(0, G, step, 0)

    out = sc_kern(xf, coef, tab)
    return out.reshape(bf, n_chunks * chunk_size)


# speedup vs baseline: 14.2986x; 14.2986x over previous
import functools

import jax
import jax.numpy as jnp
from jax import lax
from jax.experimental import pallas as pl
from jax.experimental.pallas import tpu as pltpu
from jax.experimental.pallas import tpu_sc as plsc


def kernel(x, table, hash_coeffs):
    rows, n_chunks, chunk_size = table.shape
    bf = x.shape[0] * x.shape[1]
    r_total = bf * n_chunks

    NC, NS, L = 2, 16, 16
    NW = NC * NS
    T = 2048
    per_w = r_total // NW
    G = per_w // T
    IDS = T // n_chunks

    xf = x.reshape(-1).astype(jnp.int32)
    cm = (hash_coeffs % rows).astype(jnp.int32)
    c_hi = cm // 1000
    c_lo = cm % 1000
    coef = jnp.broadcast_to(
        jnp.concatenate([c_hi, c_lo]).reshape(2 * n_chunks, 1),
        (2 * n_chunks, L)).astype(jnp.int32)
    tab = table.reshape(rows * n_chunks, chunk_size)

    mesh = plsc.VectorSubcoreMesh(core_axis_name="c", subcore_axis_name="s")

    @functools.partial(
        pl.kernel, mesh=mesh,
        out_type=jax.ShapeDtypeStruct((r_total, chunk_size), jnp.float32),
        scratch_types=[
            pltpu.VMEM((IDS,), jnp.int32),
            pltpu.VMEM((2 * n_chunks, L), jnp.int32),
            pltpu.VMEM((T,), jnp.int32),
            pltpu.VMEM((T, chunk_size), jnp.float32),
            pltpu.SemaphoreType.DMA,
        ],
        compiler_params=pltpu.CompilerParams(needs_layout_passes=False,
                                             use_tc_tiling_on_sc=False),
    )
    def sc_kern(xf_hbm, coef_hbm, tab_hbm, out_hbm,
                x_v, coef_v, idx_v, rows_v, sem):
        i32 = jnp.int32
        wid = lax.axis_index("s") * i32(NC) + lax.axis_index("c")
        pltpu.sync_copy(coef_hbm, coef_v)
        base_r = wid * i32(per_w)
        lane = lax.iota(jnp.int32, L)

        def step(g, carry):
            r0 = pl.multiple_of(base_r + g * i32(T), T)
            bf0 = pl.multiple_of(r0 // i32(n_chunks), IDS)
            pltpu.sync_copy(xf_hbm.at[pl.ds(bf0, IDS)], x_v)

            def hash_step(i, c2):
                xv = x_v[pl.ds(i * i32(L), L)]
                pos0 = i * i32(L * n_chunks) + lane * i32(n_chunks)
                for c in range(n_chunks):
                    chi_v = coef_v[c]
                    clo_v = coef_v[n_chunks + c]
                    h = ((xv * chi_v) % i32(rows) * i32(1000)
                         + xv * clo_v) % i32(rows)
                    plsc.store_scatter(idx_v, [pos0 + i32(c)],
                                       h * i32(n_chunks) + i32(c))
                return c2

            lax.fori_loop(i32(0), i32(IDS // L), hash_step, i32(0))

            copies = []
            for j in range(T // 128):
                copies.append(pltpu.async_copy(
                    tab_hbm.at[idx_v.at[pl.ds(j * 128, 128)]],
                    rows_v.at[pl.ds(j * 128, 128)],
                    sem))
            for cp in copies:
                cp.wait()
            pltpu.sync_copy(rows_v, out_hbm.at[pl.ds(r0, T)])
            return carry

        lax.fori_loop(jnp.int32(0), jnp.int32(G), step, jnp.int32(0))

    out = sc_kern(xf, coef, tab)
    return out.reshape(bf, n_chunks * chunk_size)


# speedup vs baseline: 27.6264x; 1.9321x over previous
import functools

import jax
import jax.numpy as jnp
from jax import lax
from jax.experimental import pallas as pl
from jax.experimental.pallas import tpu as pltpu
from jax.experimental.pallas import tpu_sc as plsc


def kernel(x, table, hash_coeffs):
    rows, n_chunks, chunk_size = table.shape
    bf = x.shape[0] * x.shape[1]
    r_total = bf * n_chunks

    NC, NS, L = 2, 16, 16
    NW = NC * NS
    T = 2048
    per_w = r_total // NW
    G = per_w // T
    IDS = T // n_chunks

    xf = x.reshape(-1).astype(jnp.int32)
    cm = (hash_coeffs % rows).astype(jnp.int32)
    c_hi = cm // 1000
    c_lo = cm % 1000
    coef = jnp.broadcast_to(
        jnp.concatenate([c_hi, c_lo]).reshape(2 * n_chunks, 1),
        (2 * n_chunks, L)).astype(jnp.int32)
    dim = n_chunks * chunk_size
    tab = (table.transpose(1, 2, 0).reshape(dim, rows).T
           .reshape(rows * n_chunks, chunk_size))

    mesh = plsc.VectorSubcoreMesh(core_axis_name="c", subcore_axis_name="s")

    @functools.partial(
        pl.kernel, mesh=mesh,
        out_type=jax.ShapeDtypeStruct((r_total, chunk_size), jnp.float32),
        scratch_types=[
            pltpu.VMEM((IDS,), jnp.int32),
            pltpu.VMEM((2 * n_chunks, L), jnp.int32),
            pltpu.VMEM((T,), jnp.int32),
            pltpu.VMEM((T, chunk_size), jnp.float32),
            pltpu.SemaphoreType.DMA,
        ],
        compiler_params=pltpu.CompilerParams(needs_layout_passes=False,
                                             use_tc_tiling_on_sc=False),
    )
    def sc_kern(xf_hbm, coef_hbm, tab_hbm, out_hbm,
                x_v, coef_v, idx_v, rows_v, sem):
        i32 = jnp.int32
        wid = lax.axis_index("s") * i32(NC) + lax.axis_index("c")
        pltpu.sync_copy(coef_hbm, coef_v)
        base_r = wid * i32(per_w)
        lane = lax.iota(jnp.int32, L)

        def step(g, carry):
            r0 = pl.multiple_of(base_r + g * i32(T), T)
            bf0 = pl.multiple_of(r0 // i32(n_chunks), IDS)
            pltpu.sync_copy(xf_hbm.at[pl.ds(bf0, IDS)], x_v)

            def hash_step(i, c2):
                xv = x_v[pl.ds(i * i32(L), L)]
                pos0 = i * i32(L * n_chunks) + lane * i32(n_chunks)
                for c in range(n_chunks):
                    chi_v = coef_v[c]
                    clo_v = coef_v[n_chunks + c]
                    h = ((xv * chi_v) % i32(rows) * i32(1000)
                         + xv * clo_v) % i32(rows)
                    plsc.store_scatter(idx_v, [pos0 + i32(c)],
                                       h * i32(n_chunks) + i32(c))
                return c2

            lax.fori_loop(i32(0), i32(IDS // L), hash_step, i32(0))

            copies = []
            for j in range(T // 128):
                copies.append(pltpu.async_copy(
                    tab_hbm.at[idx_v.at[pl.ds(j * 128, 128)]],
                    rows_v.at[pl.ds(j * 128, 128)],
                    sem))
            for cp in copies:
                cp.wait()
            pltpu.sync_copy(rows_v, out_hbm.at[pl.ds(r0, T)])
            return carry

        lax.fori_loop(jnp.int32(0), jnp.int32(G), step, jnp.int32(0))

    out = sc_kern(xf, coef, tab)
    return out.reshape(bf, n_chunks * chunk_size)
